# baseline (speedup 1.0000x reference)
import functools

import jax
import jax.numpy as jnp
from jax import lax
from jax.experimental import pallas as pl
from jax.experimental.pallas import tpu as pltpu
from jax.experimental.pallas import tpu_sc as plsc

S = 5000
W = 100
F = 8
D = 32

NSUB = 16
FL = S * F
CH = 2496
REM = FL - NSUB * CH
NV = CH // 16
WPAD = 112


_DNUMS = lax.GatherDimensionNumbers(offset_dims=(),
                                    collapsed_slice_dims=(0,),
                                    start_index_map=(0,))


def _gather(v, idx):
    return lax.gather(v, idx.reshape(16, 1), _DNUMS, slice_sizes=(1,),
                      mode=lax.GatherScatterMode.PROMISE_IN_BOUNDS)


def _bcast(v, lane):
    return _gather(v, jnp.full((16,), lane, jnp.int32))


def _treesum(v):
    lanes = lax.iota(jnp.int32, 16)
    for sh in (1, 2, 4, 8):
        v = v + _gather(v, lanes ^ sh)
    return v


def _policy_sc_body(feats_hbm, we_hbm, be_hbm, w1_hbm, b1_hbm, w2_hbm,
                    b2_hbm, wd_hbm, bd_hbm,
                    out_hbm, part_hbm,
                    buf, rembuf, stage, partv, wev, bev, w1v, b1v, w2v, b2v,
                    wdv, bdv, outv):
    sid = lax.axis_index("s") + lax.axis_index("c") * NSUB
    base = sid * CH
    pltpu.sync_copy(feats_hbm.at[pl.ds(base, CH)], buf)

    def body(i, a):
        return a + buf[pl.ds(i * 16, 16)]

    acc = lax.fori_loop(0, NV, body, jnp.zeros((16,), jnp.float32))

    pltpu.sync_copy(feats_hbm.at[pl.ds(NSUB * CH, REM)], rembuf)
    racc = jnp.zeros((16,), jnp.float32)
    for i in range(REM // 16):
        racc = racc + rembuf[pl.ds(i * 16, 16)]
    keep = jnp.where(sid == 0, 1.0, 0.0).astype(jnp.float32)
    stage[...] = acc + racc * keep
    pltpu.sync_copy(stage, part_hbm.at[sid])
    plsc.subcore_barrier()

    @pl.when(sid == 0)
    def _tail():
        pltpu.sync_copy(part_hbm, partv)
        pltpu.sync_copy(we_hbm, wev)
        pltpu.sync_copy(be_hbm, bev)
        pltpu.sync_copy(w1_hbm, w1v)
        pltpu.sync_copy(b1_hbm, b1v)
        pltpu.sync_copy(w2_hbm, w2v)
        pltpu.sync_copy(b2_hbm, b2v)
        pltpu.sync_copy(wd_hbm, wdv)
        pltpu.sync_copy(bd_hbm, bdv)

        total = jnp.zeros((16,), jnp.float32)
        for i in range(NSUB):
            total = total + partv[i]

        inv_s = 1.0 / S
        mean_b = [(_bcast(total, f) + _bcast(total, f + F)) * inv_s
                  for f in range(F)]

        mw = [bev[pl.ds(h * 16, 16)] for h in range(2)]
        for f in range(F):
            for h in range(2):
                mw[h] = mw[h] + mean_b[f] * wev[f, pl.ds(h * 16, 16)]

        h1w = [b1v[pl.ds(h * 16, 16)] for h in range(2)]
        for f in range(D):
            mb = _bcast(mw[f // 16], f % 16)
            for h in range(2):
                h1w[h] = h1w[h] + mb * w1v[f, pl.ds(h * 16, 16)]
        h1w = [jnp.maximum(v, 0.0) for v in h1w]
        h1s = [jnp.maximum(b1v[pl.ds(h * 16, 16)], 0.0) for h in range(2)]

        h2s = [b2v[pl.ds(h * 16, 16)] for h in range(2)]
        h2w = [b2v[pl.ds(h * 16, 16)] for h in range(2)]
        for f in range(D):
            sb = _bcast(h1w[f // 16], f % 16)
            wb = _bcast(h1s[f // 16], f % 16)
            for h in range(2):
                row = w2v[f, pl.ds(h * 16, 16)]
                h2s[h] = h2s[h] + sb * row
                h2w[h] = h2w[h] + wb * row

        partial = (h2s[0] * wdv[pl.ds(0, 16)] + h2s[1] * wdv[pl.ds(16, 16)]
                   + h2w[0] * wdv[pl.ds(32, 16)] + h2w[1] * wdv[pl.ds(48, 16)])
        score = _treesum(partial) + _bcast(bdv[...], 0)

        ev = jnp.exp(score - score)
        lastmask = jnp.where(lax.iota(jnp.int32, 16) < (W % 16), 1.0, 0.0)
        denom = float(W // 16) * _treesum(ev) + _treesum(ev * lastmask)
        p = ev / denom
        for i in range(WPAD // 16):
            outv[pl.ds(i * 16, 16)] = p
        pltpu.sync_copy(outv, out_hbm)


def kernel(state, W_embed, b_embed, W1, b1, W2, b2, W_dec, b_dec):
    feats = state[:, :F].reshape(FL)
    bd16 = jnp.pad(b_dec, (0, 15))
    mesh = plsc.VectorSubcoreMesh(core_axis_name="c", subcore_axis_name="s",
                                  num_cores=1, num_subcores=NSUB)
    policy = functools.partial(
        pl.kernel,
        out_type=(jax.ShapeDtypeStruct((WPAD,), jnp.float32),
                  jax.ShapeDtypeStruct((NSUB, 16), jnp.float32)),
        mesh=mesh,
        scratch_types=[
            pltpu.VMEM((CH,), jnp.float32),
            pltpu.VMEM((REM,), jnp.float32),
            pltpu.VMEM((16,), jnp.float32),
            pltpu.VMEM((NSUB, 16), jnp.float32),
            pltpu.VMEM((F, D), jnp.float32),
            pltpu.VMEM((D,), jnp.float32),
            pltpu.VMEM((D, D), jnp.float32),
            pltpu.VMEM((D,), jnp.float32),
            pltpu.VMEM((D, D), jnp.float32),
            pltpu.VMEM((D,), jnp.float32),
            pltpu.VMEM((2 * D,), jnp.float32),
            pltpu.VMEM((16,), jnp.float32),
            pltpu.VMEM((WPAD,), jnp.float32),
        ],
    )(_policy_sc_body)
    out, _ = policy(feats, W_embed, b_embed, W1, b1, W2, b2,
                    W_dec.reshape(2 * D), bd16)
    return out[:W]


# speedup vs baseline: 1.1547x; 1.1547x over previous
import functools

import jax
import jax.numpy as jnp
from jax import lax
from jax.experimental import pallas as pl
from jax.experimental.pallas import tpu as pltpu
from jax.experimental.pallas import tpu_sc as plsc

S = 5000
W = 100
F = 8
D = 32

NSUB = 16
FL = S * F
CH = 2496
REM = FL - NSUB * CH
NV = CH // 16
WPAD = 112

O_WE = 0
O_BE = O_WE + F * D
O_W1 = O_BE + D
O_B1 = O_W1 + D * D
O_W2 = O_B1 + D
O_B2 = O_W2 + D * D
O_WD = O_B2 + D
O_BD = O_WD + 2 * D
WLEN = O_BD + 16

_DNUMS = lax.GatherDimensionNumbers(offset_dims=(),
                                    collapsed_slice_dims=(0,),
                                    start_index_map=(0,))


def _gather(v, idx):
    return lax.gather(v, idx.reshape(16, 1), _DNUMS, slice_sizes=(1,),
                      mode=lax.GatherScatterMode.PROMISE_IN_BOUNDS)


def _bcast(v, lane):
    return _gather(v, jnp.full((16,), lane, jnp.int32))


def _treesum(v):
    lanes = lax.iota(jnp.int32, 16)
    for sh in (1, 2, 4, 8):
        v = v + _gather(v, lanes ^ sh)
    return v


def _policy_sc_body(feats_hbm, wflat_hbm, out_hbm, part_hbm,
                    buf, rembuf, stage, partv, wv, outv, sem):
    sid = lax.axis_index("s") + lax.axis_index("c") * NSUB

    wcopy = pltpu.async_copy(wflat_hbm, wv, sem)

    base = sid * CH
    pltpu.sync_copy(feats_hbm.at[pl.ds(base, CH)], buf)

    def body(i, a):
        return a + buf[pl.ds(i * 16, 16)]

    acc = lax.fori_loop(0, NV, body, jnp.zeros((16,), jnp.float32))

    pltpu.sync_copy(feats_hbm.at[pl.ds(NSUB * CH, REM)], rembuf)
    racc = jnp.zeros((16,), jnp.float32)
    for i in range(REM // 16):
        racc = racc + rembuf[pl.ds(i * 16, 16)]
    keep = jnp.where(sid == 0, 1.0, 0.0).astype(jnp.float32)
    stage[...] = acc + racc * keep
    pltpu.sync_copy(stage, part_hbm.at[sid])
    wcopy.wait()
    plsc.subcore_barrier()

    @pl.when(sid == 0)
    def _tail():
        pltpu.sync_copy(part_hbm, partv)

        total = jnp.zeros((16,), jnp.float32)
        for i in range(NSUB):
            total = total + partv[i]

        def wrow(off):
            return wv[pl.ds(off, 16)]

        inv_s = 1.0 / S
        mean_b = [(_bcast(total, f) + _bcast(total, f + F)) * inv_s
                  for f in range(F)]

        mw = [wrow(O_BE + h * 16) for h in range(2)]
        for f in range(F):
            for h in range(2):
                mw[h] = mw[h] + mean_b[f] * wrow(O_WE + f * D + h * 16)

        h1w = [wrow(O_B1 + h * 16) for h in range(2)]
        for f in range(D):
            mb = _bcast(mw[f // 16], f % 16)
            for h in range(2):
                h1w[h] = h1w[h] + mb * wrow(O_W1 + f * D + h * 16)
        h1w = [jnp.maximum(v, 0.0) for v in h1w]
        h1s = [jnp.maximum(wrow(O_B1 + h * 16), 0.0) for h in range(2)]

        h2s = [wrow(O_B2 + h * 16) for h in range(2)]
        h2w = [wrow(O_B2 + h * 16) for h in range(2)]
        for f in range(D):
            sb = _bcast(h1w[f // 16], f % 16)
            wb = _bcast(h1s[f // 16], f % 16)
            for h in range(2):
                row = wrow(O_W2 + f * D + h * 16)
                h2s[h] = h2s[h] + sb * row
                h2w[h] = h2w[h] + wb * row

        partial = (h2s[0] * wrow(O_WD) + h2s[1] * wrow(O_WD + 16)
                   + h2w[0] * wrow(O_WD + 32) + h2w[1] * wrow(O_WD + 48))
        score = _treesum(partial) + _bcast(wrow(O_BD), 0)

        ev = jnp.exp(score - score)
        lastmask = jnp.where(lax.iota(jnp.int32, 16) < (W % 16), 1.0, 0.0)
        denom = float(W // 16) * _treesum(ev) + _treesum(ev * lastmask)
        p = ev / denom
        for i in range(WPAD // 16):
            outv[pl.ds(i * 16, 16)] = p
        pltpu.sync_copy(outv, out_hbm)


def kernel(state, W_embed, b_embed, W1, b1, W2, b2, W_dec, b_dec):
    feats = state[:, :F].reshape(FL)
    wflat = jnp.concatenate([
        W_embed.reshape(F * D), b_embed,
        W1.reshape(D * D), b1,
        W2.reshape(D * D), b2,
        W_dec.reshape(2 * D), jnp.pad(b_dec, (0, 15)),
    ])
    mesh = plsc.VectorSubcoreMesh(core_axis_name="c", subcore_axis_name="s",
                                  num_cores=1, num_subcores=NSUB)
    policy = functools.partial(
        pl.kernel,
        out_type=(jax.ShapeDtypeStruct((WPAD,), jnp.float32),
                  jax.ShapeDtypeStruct((NSUB, 16), jnp.float32)),
        mesh=mesh,
        scratch_types=[
            pltpu.VMEM((CH,), jnp.float32),
            pltpu.VMEM((REM,), jnp.float32),
            pltpu.VMEM((16,), jnp.float32),
            pltpu.VMEM((NSUB, 16), jnp.float32),
            pltpu.VMEM((WLEN,), jnp.float32),
            pltpu.VMEM((WPAD,), jnp.float32),
            pltpu.SemaphoreType.DMA,
        ],
    )(_policy_sc_body)
    out, _ = policy(feats, wflat)
    return out[:W]


# speedup vs baseline: 1.2878x; 1.1152x over previous
import functools

import jax
import jax.numpy as jnp
from jax import lax
from jax.experimental import pallas as pl
from jax.experimental.pallas import tpu as pltpu
from jax.experimental.pallas import tpu_sc as plsc

S = 5000
W = 100
F = 8
D = 32

NSUB = 16
FL = S * F
CH = 2496
REM = FL - NSUB * CH
NV = CH // 16
UNROLL = 4
WPAD = 112

O_WE = 0
O_BE = O_WE + F * D
O_W1 = O_BE + D
O_B1 = O_W1 + D * D
O_W2 = O_B1 + D
O_B2 = O_W2 + D * D
O_WD = O_B2 + D
WLEN = O_WD + 2 * D

_DNUMS = lax.GatherDimensionNumbers(offset_dims=(),
                                    collapsed_slice_dims=(0,),
                                    start_index_map=(0,))


def _gather(v, idx):
    return lax.gather(v, idx.reshape(16, 1), _DNUMS, slice_sizes=(1,),
                      mode=lax.GatherScatterMode.PROMISE_IN_BOUNDS)


def _bcast(v, lane):
    return _gather(v, jnp.full((16,), lane, jnp.int32))


def _treesum(v):
    lanes = lax.iota(jnp.int32, 16)
    for sh in (1, 2, 4, 8):
        v = v + _gather(v, lanes ^ sh)
    return v


def _policy_sc_body(feats_hbm, wflat_hbm, out_hbm,
                    buf, rembuf, stage, partv, wv, outv, shared, sem):
    sid = lax.axis_index("s") + lax.axis_index("c") * NSUB

    copies = []

    @pl.when(sid == 0)
    def _prefetch():
        copies.append(pltpu.async_copy(wflat_hbm, wv, sem))
        copies.append(pltpu.async_copy(
            feats_hbm.at[pl.ds(NSUB * CH, REM)], rembuf, sem))

    pltpu.sync_copy(feats_hbm.at[pl.ds(sid * CH, CH)], buf)

    def body(i, accs):
        return tuple(a + buf[pl.ds((i * UNROLL + k) * 16, 16)]
                     for k, a in enumerate(accs))

    accs = lax.fori_loop(0, NV // UNROLL, body,
                         tuple(jnp.zeros((16,), jnp.float32)
                               for _ in range(UNROLL)))
    stage[...] = functools.reduce(lambda a, b: a + b, accs)
    pltpu.sync_copy(stage, shared.at[sid])

    plsc.subcore_barrier()

    @pl.when(sid == 0)
    def _tail():
        for c in copies:
            c.wait()
        pltpu.sync_copy(shared, partv)

        total = jnp.zeros((16,), jnp.float32)
        for i in range(NSUB):
            total = total + partv[i]
        for i in range(REM // 16):
            total = total + rembuf[pl.ds(i * 16, 16)]

        def wrow(off):
            return wv[pl.ds(off, 16)]

        inv_s = 1.0 / S
        mean_b = [(_bcast(total, f) + _bcast(total, f + F)) * inv_s
                  for f in range(F)]

        mw = [wrow(O_BE + h * 16) for h in range(2)]
        for f in range(F):
            for h in range(2):
                mw[h] = mw[h] + mean_b[f] * wrow(O_WE + f * D + h * 16)

        h1w = [wrow(O_B1 + h * 16) for h in range(2)]
        for f in range(D):
            mb = _bcast(mw[f // 16], f % 16)
            for h in range(2):
                h1w[h] = h1w[h] + mb * wrow(O_W1 + f * D + h * 16)
        h1w = [jnp.maximum(v, 0.0) for v in h1w]
        h1s = [jnp.maximum(wrow(O_B1 + h * 16), 0.0) for h in range(2)]

        h2s = [wrow(O_B2 + h * 16) for h in range(2)]
        h2w = [wrow(O_B2 + h * 16) for h in range(2)]
        for f in range(D):
            sb = _bcast(h1w[f // 16], f % 16)
            wb = _bcast(h1s[f // 16], f % 16)
            for h in range(2):
                row = wrow(O_W2 + f * D + h * 16)
                h2s[h] = h2s[h] + sb * row
                h2w[h] = h2w[h] + wb * row

        partial = (h2s[0] * wrow(O_WD) + h2s[1] * wrow(O_WD + 16)
                   + h2w[0] * wrow(O_WD + 32) + h2w[1] * wrow(O_WD + 48))
        score = _treesum(partial)

        ev = jnp.exp(score - score)
        lastmask = jnp.where(lax.iota(jnp.int32, 16) < (W % 16), 1.0, 0.0)
        denom = float(W // 16) * _treesum(ev) + _treesum(ev * lastmask)
        p = ev / denom
        for i in range(WPAD // 16):
            outv[pl.ds(i * 16, 16)] = p
        pltpu.sync_copy(outv, out_hbm)


def kernel(state, W_embed, b_embed, W1, b1, W2, b2, W_dec, b_dec):
    del b_dec
    feats = state[:, :F].reshape(FL)
    wflat = jnp.concatenate([
        W_embed.reshape(F * D), b_embed,
        W1.reshape(D * D), b1,
        W2.reshape(D * D), b2,
        W_dec.reshape(2 * D),
    ])
    mesh = plsc.VectorSubcoreMesh(core_axis_name="c", subcore_axis_name="s",
                                  num_cores=1, num_subcores=NSUB)
    policy = functools.partial(
        pl.kernel,
        out_type=jax.ShapeDtypeStruct((WPAD,), jnp.float32),
        mesh=mesh,
        scratch_types=[
            pltpu.VMEM((CH,), jnp.float32),
            pltpu.VMEM((REM,), jnp.float32),
            pltpu.VMEM((16,), jnp.float32),
            pltpu.VMEM((NSUB, 16), jnp.float32),
            pltpu.VMEM((WLEN,), jnp.float32),
            pltpu.VMEM((WPAD,), jnp.float32),
            pltpu.VMEM_SHARED((NSUB, 16), jnp.float32),
            pltpu.SemaphoreType.DMA,
        ],
    )(_policy_sc_body)
    out = policy(feats, wflat)
    return out[:W]


# speedup vs baseline: 1.3243x; 1.0284x over previous
import functools

import jax
import jax.numpy as jnp
from jax import lax
from jax.experimental import pallas as pl
from jax.experimental.pallas import tpu as pltpu
from jax.experimental.pallas import tpu_sc as plsc

S = 5000
W = 100
F = 8
D = 32

NSUB = 16
FL = S * F
CH = 2496
REM = FL - NSUB * CH
NV = CH // 16
UNROLL = 4
WPAD = 112

O_WE = 0
O_BE = O_WE + F * D
O_W1 = O_BE + D
O_B1 = O_W1 + D * D
O_W2 = O_B1 + D
O_B2 = O_W2 + D * D
O_WD = O_B2 + D
WLEN = O_WD + 2 * D

_DNUMS = lax.GatherDimensionNumbers(offset_dims=(),
                                    collapsed_slice_dims=(0,),
                                    start_index_map=(0,))


def _gather(v, idx):
    return lax.gather(v, idx.reshape(16, 1), _DNUMS, slice_sizes=(1,),
                      mode=lax.GatherScatterMode.PROMISE_IN_BOUNDS)


def _bcast(v, lane):
    return _gather(v, jnp.full((16,), lane, jnp.int32))


def _treesum(v):
    lanes = lax.iota(jnp.int32, 16)
    for sh in (1, 2, 4, 8):
        v = v + _gather(v, lanes ^ sh)
    return v


def _policy_sc_body(packed_hbm, out_hbm,
                    buf, rembuf, stage, partv, wv, outv, shared, sem):
    sid = lax.axis_index("s") + lax.axis_index("c") * NSUB

    copies = []

    @pl.when(sid == 0)
    def _prefetch():
        copies.append(pltpu.async_copy(
            packed_hbm.at[pl.ds(FL, WLEN)], wv, sem))
        copies.append(pltpu.async_copy(
            packed_hbm.at[pl.ds(NSUB * CH, REM)], rembuf, sem))

    pltpu.sync_copy(packed_hbm.at[pl.ds(sid * CH, CH)], buf)

    def body(i, accs):
        return tuple(a + buf[pl.ds((i * UNROLL + k) * 16, 16)]
                     for k, a in enumerate(accs))

    accs = lax.fori_loop(0, NV // UNROLL, body,
                         tuple(jnp.zeros((16,), jnp.float32)
                               for _ in range(UNROLL)))
    stage[...] = functools.reduce(lambda a, b: a + b, accs)
    pltpu.sync_copy(stage, shared.at[sid])

    plsc.subcore_barrier()

    @pl.when(sid == 0)
    def _tail():
        for c in copies:
            c.wait()
        pltpu.sync_copy(shared, partv)

        total = jnp.zeros((16,), jnp.float32)
        for i in range(NSUB):
            total = total + partv[i]
        for i in range(REM // 16):
            total = total + rembuf[pl.ds(i * 16, 16)]

        def wrow(off):
            return wv[pl.ds(off, 16)]

        inv_s = 1.0 / S
        mean_b = [(_bcast(total, f) + _bcast(total, f + F)) * inv_s
                  for f in range(F)]

        mw = [wrow(O_BE + h * 16) for h in range(2)]
        for f in range(F):
            for h in range(2):
                mw[h] = mw[h] + mean_b[f] * wrow(O_WE + f * D + h * 16)

        h1w = [wrow(O_B1 + h * 16) for h in range(2)]
        for f in range(D):
            mb = _bcast(mw[f // 16], f % 16)
            for h in range(2):
                h1w[h] = h1w[h] + mb * wrow(O_W1 + f * D + h * 16)
        h1w = [jnp.maximum(v, 0.0) for v in h1w]
        h1s = [jnp.maximum(wrow(O_B1 + h * 16), 0.0) for h in range(2)]

        h2s = [wrow(O_B2 + h * 16) for h in range(2)]
        h2w = [wrow(O_B2 + h * 16) for h in range(2)]
        for f in range(D):
            sb = _bcast(h1w[f // 16], f % 16)
            wb = _bcast(h1s[f // 16], f % 16)
            for h in range(2):
                row = wrow(O_W2 + f * D + h * 16)
                h2s[h] = h2s[h] + sb * row
                h2w[h] = h2w[h] + wb * row

        partial = (h2s[0] * wrow(O_WD) + h2s[1] * wrow(O_WD + 16)
                   + h2w[0] * wrow(O_WD + 32) + h2w[1] * wrow(O_WD + 48))
        score = _treesum(partial)

        ev = jnp.exp(score - score)
        lastmask = jnp.where(lax.iota(jnp.int32, 16) < (W % 16), 1.0, 0.0)
        denom = float(W // 16) * _treesum(ev) + _treesum(ev * lastmask)
        p = ev / denom
        for i in range(WPAD // 16):
            outv[pl.ds(i * 16, 16)] = p
        pltpu.sync_copy(outv, out_hbm)


def kernel(state, W_embed, b_embed, W1, b1, W2, b2, W_dec, b_dec):
    del b_dec
    packed = jnp.concatenate([
        state[:, :F].reshape(FL),
        W_embed.reshape(F * D), b_embed,
        W1.reshape(D * D), b1,
        W2.reshape(D * D), b2,
        W_dec.reshape(2 * D),
    ])
    mesh = plsc.VectorSubcoreMesh(core_axis_name="c", subcore_axis_name="s",
                                  num_cores=1, num_subcores=NSUB)
    policy = functools.partial(
        pl.kernel,
        out_type=jax.ShapeDtypeStruct((WPAD,), jnp.float32),
        mesh=mesh,
        scratch_types=[
            pltpu.VMEM((CH,), jnp.float32),
            pltpu.VMEM((REM,), jnp.float32),
            pltpu.VMEM((16,), jnp.float32),
            pltpu.VMEM((NSUB, 16), jnp.float32),
            pltpu.VMEM((WLEN,), jnp.float32),
            pltpu.VMEM((WPAD,), jnp.float32),
            pltpu.VMEM_SHARED((NSUB, 16), jnp.float32),
            pltpu.SemaphoreType.DMA,
        ],
    )(_policy_sc_body)
    out = policy(packed)
    return out[:W]


# speedup vs baseline: 1.3316x; 1.0055x over previous
import functools

import jax
import jax.numpy as jnp
from jax import lax
from jax.experimental import pallas as pl
from jax.experimental.pallas import tpu as pltpu
from jax.experimental.pallas import tpu_sc as plsc

S = 5000
W = 100
F = 8
D = 32

NSUB = 16
FL = S * F
CH = 2496
REM = FL - NSUB * CH
NV = CH // 16
UNROLL = 4
WPAD = 112

L_BE = 0
L_B1 = L_BE + D
L_B2 = L_B1 + D
L_WD = L_B2 + D
WLEN = L_WD + 2 * D

_DNUMS = lax.GatherDimensionNumbers(offset_dims=(),
                                    collapsed_slice_dims=(0,),
                                    start_index_map=(0,))


def _gather(v, idx):
    return lax.gather(v, idx.reshape(16, 1), _DNUMS, slice_sizes=(1,),
                      mode=lax.GatherScatterMode.PROMISE_IN_BOUNDS)


def _bcast(v, lane):
    return _gather(v, jnp.full((16,), lane, jnp.int32))


def _treesum(v):
    lanes = lax.iota(jnp.int32, 16)
    for sh in (1, 2, 4, 8):
        v = v + _gather(v, lanes ^ sh)
    return v


def _policy_sc_body(packed_hbm, we_hbm, w1_hbm, w2_hbm, out_hbm,
                    buf, rembuf, stage, partv, wv, web, w1b, w2b,
                    outv, shared, sem):
    sid = lax.axis_index("s") + lax.axis_index("c") * NSUB

    copies = []

    @pl.when(sid == 0)
    def _prefetch():
        copies.append(pltpu.async_copy(
            packed_hbm.at[pl.ds(FL, WLEN)], wv, sem))
        copies.append(pltpu.async_copy(we_hbm, web, sem))
        copies.append(pltpu.async_copy(w1_hbm, w1b, sem))
        copies.append(pltpu.async_copy(w2_hbm, w2b, sem))
        copies.append(pltpu.async_copy(
            packed_hbm.at[pl.ds(NSUB * CH, REM)], rembuf, sem))

    pltpu.sync_copy(packed_hbm.at[pl.ds(sid * CH, CH)], buf)

    def body(i, accs):
        return tuple(a + buf[pl.ds((i * UNROLL + k) * 16, 16)]
                     for k, a in enumerate(accs))

    accs = lax.fori_loop(0, NV // UNROLL, body,
                         tuple(jnp.zeros((16,), jnp.float32)
                               for _ in range(UNROLL)))
    stage[...] = functools.reduce(lambda a, b: a + b, accs)
    pltpu.sync_copy(stage, shared.at[sid])

    plsc.subcore_barrier()

    @pl.when(sid == 0)
    def _tail():
        for c in copies:
            c.wait()
        pltpu.sync_copy(shared, partv)

        total = jnp.zeros((16,), jnp.float32)
        for i in range(NSUB):
            total = total + partv[i]
        for i in range(REM // 16):
            total = total + rembuf[pl.ds(i * 16, 16)]

        def wrow(off):
            return wv[pl.ds(off, 16)]

        inv_s = 1.0 / S
        mean_b = [(_bcast(total, f) + _bcast(total, f + F)) * inv_s
                  for f in range(F)]

        mw = [wrow(L_BE + h * 16) for h in range(2)]
        for f in range(F):
            for h in range(2):
                mw[h] = mw[h] + mean_b[f] * web[f, pl.ds(h * 16, 16)]

        h1w = [wrow(L_B1 + h * 16) for h in range(2)]
        for f in range(D):
            mb = _bcast(mw[f // 16], f % 16)
            for h in range(2):
                h1w[h] = h1w[h] + mb * w1b[f, pl.ds(h * 16, 16)]
        h1w = [jnp.maximum(v, 0.0) for v in h1w]
        h1s = [jnp.maximum(wrow(L_B1 + h * 16), 0.0) for h in range(2)]

        h2s = [wrow(L_B2 + h * 16) for h in range(2)]
        h2w = [wrow(L_B2 + h * 16) for h in range(2)]
        for f in range(D):
            sb = _bcast(h1w[f // 16], f % 16)
            wb = _bcast(h1s[f // 16], f % 16)
            for h in range(2):
                row = w2b[f, pl.ds(h * 16, 16)]
                h2s[h] = h2s[h] + sb * row
                h2w[h] = h2w[h] + wb * row

        partial = (h2s[0] * wrow(L_WD) + h2s[1] * wrow(L_WD + 16)
                   + h2w[0] * wrow(L_WD + 32) + h2w[1] * wrow(L_WD + 48))
        score = _treesum(partial)

        ev = jnp.exp(score - score)
        lastmask = jnp.where(lax.iota(jnp.int32, 16) < (W % 16), 1.0, 0.0)
        denom = float(W // 16) * _treesum(ev) + _treesum(ev * lastmask)
        p = ev / denom
        for i in range(WPAD // 16):
            outv[pl.ds(i * 16, 16)] = p
        pltpu.sync_copy(outv, out_hbm)


def kernel(state, W_embed, b_embed, W1, b1, W2, b2, W_dec, b_dec):
    del b_dec
    packed = jnp.concatenate([
        state[:, :F].reshape(FL),
        b_embed, b1, b2, W_dec.reshape(2 * D),
    ])
    mesh = plsc.VectorSubcoreMesh(core_axis_name="c", subcore_axis_name="s",
                                  num_cores=1, num_subcores=NSUB)
    policy = functools.partial(
        pl.kernel,
        out_type=jax.ShapeDtypeStruct((WPAD,), jnp.float32),
        mesh=mesh,
        scratch_types=[
            pltpu.VMEM((CH,), jnp.float32),
            pltpu.VMEM((REM,), jnp.float32),
            pltpu.VMEM((16,), jnp.float32),
            pltpu.VMEM((NSUB, 16), jnp.float32),
            pltpu.VMEM((WLEN,), jnp.float32),
            pltpu.VMEM((F, D), jnp.float32),
            pltpu.VMEM((D, D), jnp.float32),
            pltpu.VMEM((D, D), jnp.float32),
            pltpu.VMEM((WPAD,), jnp.float32),
            pltpu.VMEM_SHARED((NSUB, 16), jnp.float32),
            pltpu.SemaphoreType.DMA,
        ],
    )(_policy_sc_body)
    out = policy(packed, W_embed, W1, W2)
    return out[:W]
